# baseline (speedup 1.0000x reference)
import jax
import jax.numpy as jnp
from jax import lax
from jax.experimental import pallas as pl
from jax.experimental.pallas import tpu as pltpu, tpu_sc as plsc

_NC = 2
_NS = 16
_CH = 32768


def kernel(x, demog_label, att_channel):
    B, C, H, W = x.shape
    nd = att_channel.shape[0]
    att2 = att_channel.reshape(nd, C)
    CHW = C * H * W
    x2 = x.reshape(B, CHW)

    nw = _NC * _NS
    rows_per_w = B // nw
    kch = CHW // _CH
    nchunks = rows_per_w * kch
    n_active = 16
    b_per_w = B // n_active

    mesh = plsc.VectorSubcoreMesh(core_axis_name="c", subcore_axis_name="s")

    def _sc_body(x_hbm, att_hbm, lab_hbm, y_hbm, att_out_hbm,
                 idx_v, rows_v, att_v, buf0, buf1, gsem,
                 si0, si1, so0, so1):
        wid = lax.axis_index("s") * _NC + lax.axis_index("c")
        base = wid * rows_per_w
        bufs = (buf0, buf1)
        sin = (si0, si1)
        sout = (so0, so1)

        def chunk_refs(i):
            row = base + i // kch
            off = (i % kch) * _CH
            return x_hbm.at[row, pl.ds(off, _CH)], y_hbm.at[row, pl.ds(off, _CH)]

        src0, _ = chunk_refs(0)
        cp_in = [None] * nchunks
        cp_out = [None] * nchunks
        cp_in[0] = pltpu.async_copy(src0, bufs[0], sin[0])

        @pl.when(wid < n_active)
        def _gather():
            gb = wid * b_per_w
            pltpu.sync_copy(lab_hbm.at[pl.ds(gb, b_per_w)], idx_v)
            pltpu.async_copy(att_hbm.at[idx_v], rows_v, gsem).wait()

        @pl.when(wid == n_active)
        def _att_copy():
            pltpu.sync_copy(att_hbm, att_v)
            pltpu.sync_copy(att_v, att_out_hbm)

        for i in range(nchunks):
            b = i % 2
            cp_in[i].wait()
            if i + 1 < nchunks:
                nb = (i + 1) % 2
                if i - 1 >= 0:
                    cp_out[i - 1].wait()
                nsrc, _ = chunk_refs(i + 1)
                cp_in[i + 1] = pltpu.async_copy(nsrc, bufs[nb], sin[nb])
            _, dst = chunk_refs(i)
            cp_out[i] = pltpu.async_copy(bufs[b], dst, sout[b])
        cp_out[nchunks - 1].wait()
        if nchunks >= 2:
            cp_out[nchunks - 2].wait()

    sc_call = pl.kernel(
        _sc_body,
        out_type=[
            jax.ShapeDtypeStruct((B, CHW), jnp.float32),
            jax.ShapeDtypeStruct((nd, C), jnp.float32),
        ],
        mesh=mesh,
        scratch_types=[
            pltpu.VMEM((b_per_w,), jnp.int32),
            pltpu.VMEM((b_per_w, C), jnp.float32),
            pltpu.VMEM((nd, C), jnp.float32),
            pltpu.VMEM((_CH,), jnp.float32),
            pltpu.VMEM((_CH,), jnp.float32),
            pltpu.SemaphoreType.DMA,
            pltpu.SemaphoreType.DMA,
            pltpu.SemaphoreType.DMA,
            pltpu.SemaphoreType.DMA,
            pltpu.SemaphoreType.DMA,
        ],
        name="att_row_gather_sc",
    )
    y2, att_out = sc_call(x2, att2, demog_label)

    return (y2.reshape(B, C, H, W), att_out.reshape(att_channel.shape))
